# baseline (speedup 1.0000x reference)
import functools

import jax
import jax.numpy as jnp
from jax import lax
from jax.experimental import pallas as pl
from jax.experimental.pallas import tpu as pltpu
from jax.experimental.pallas import tpu_sc as plsc

_SPARSE_RATIO = 0.5
_CL = 512


def _norm_body(x_ref, nb_ref, nbcol_ref, *, L: int):
    nc = L // _CL
    rows = _CL // 128
    for i in range(nc):
        part = x_ref[0, pl.ds(i * _CL, _CL), :]
        s = jnp.sum(part * part, axis=-1)
        nbcol_ref[0, pl.ds(i * _CL, _CL), :] = (
            jax.lax.bitcast_convert_type(s, jnp.int32)[:, None])
        s2 = s.reshape(rows, 128)
        nb_ref[0, pl.ds(i * rows, rows), :] = (
            jax.lax.bitcast_convert_type(jnp.sqrt(s2), jnp.int32))


def _sc_body(nb_hbm, sq_hbm, aux_hbm, nbuf, sqbuf, auxbuf, *, K: int, L: int,
             B: int, NC: int):
    wid = lax.axis_index("s") * NC + lax.axis_index("c")
    nv = L // 16

    one = jnp.full((16,), 1.0, jnp.float32)
    zero = jnp.zeros((16,), jnp.float32)
    lane15 = jnp.full((16,), 15, jnp.int32)
    Kf = jnp.full((16,), float(K), jnp.float32)

    io = lax.iota(jnp.int32, 16)

    def _perm(vec, shift):
        idx = jnp.bitwise_and(io + shift, 15)
        return lax.gather(
            vec, idx[:, None],
            lax.GatherDimensionNumbers(offset_dims=(),
                                       collapsed_slice_dims=(0,),
                                       start_index_map=(0,)),
            (1,), mode=lax.GatherScatterMode.PROMISE_IN_BOUNDS)

    def _tree(vec, op):
        for sh in (8, 4, 2, 1):
            vec = op(vec, _perm(vec, sh))
        return vec

    def _splat_total(acc):
        return _tree(acc, jnp.add)

    @pl.when(wid < B)
    def _work():
        pltpu.sync_copy(nb_hbm.at[wid], nbuf)
        pltpu.sync_copy(sq_hbm.at[wid], sqbuf)

        def outer(i, t_vec):
            bit = jnp.full((16,), 1, jnp.int32) << jnp.broadcast_to(30 - i, (16,))
            cand = jnp.bitwise_or(t_vec, bit)

            def inner(v, acc):
                bits = nbuf[pl.ds(v * 16, 16)]
                return acc + jnp.where(bits >= cand, one, zero)

            accv = lax.fori_loop(0, nv, inner, zero)
            cnt = _splat_total(accv)
            return jnp.where(cnt >= Kf, cand, t_vec)

        T = lax.fori_loop(0, 31, outer, jnp.zeros((16,), jnp.int32))

        def stats(v, carry):
            accg, acclo, acchi = carry
            bits = nbuf[pl.ds(v * 16, 16)]
            sqf = sqbuf[pl.ds(v * 16, 16)]
            iseq = bits == T
            accg = accg + jnp.where(bits > T, one, zero)
            acclo = jnp.where(iseq, jnp.minimum(acclo, sqf), acclo)
            acchi = jnp.where(iseq, jnp.maximum(acchi, sqf), acchi)
            return accg, acclo, acchi

        accg, acclo, acchi = lax.fori_loop(
            0, nv, stats,
            (zero,
             jnp.full((16,), jnp.inf, jnp.float32),
             jnp.full((16,), -jnp.inf, jnp.float32)))
        g = _splat_total(accg)
        hi = _tree(acchi, jnp.maximum)
        lo = _tree(acclo, jnp.minimum)
        m = Kf - g

        v0 = jnp.where(io == 0, lo, jnp.where(io == 1, hi,
                       jnp.where(io == 2, m, zero)))
        v1 = zero
        nvc = _CL // 16
        pre = zero
        for cc in range(L // _CL):
            slot = 3 + cc
            v0 = jnp.where(io == slot, pre, v0) if slot < 16 else v0
            v1 = jnp.where(io == slot - 16, pre, v1) if slot >= 16 else v1

            def ceq(v, acc):
                bits = nbuf[pl.ds((cc * nvc + v) * 16, 16)]
                return acc + jnp.where(bits == T, one, zero)

            acce = lax.fori_loop(0, nvc, ceq, zero)
            pre = pre + _splat_total(acce)
        auxbuf[pl.ds(0, 16)] = v0
        auxbuf[pl.ds(16, 16)] = v1
        pltpu.sync_copy(auxbuf, aux_hbm.at[wid])


def _apply_body(x_ref, bc_ref, aux_ref, ltri_ref, o_ref):
    c = pl.program_id(1)
    lo = aux_ref[0, 0, 0]
    hi = aux_ref[0, 0, 1]
    m = aux_ref[0, 0, 2]
    pre = aux_ref[0, 0, 3 + c]

    bcf = jax.lax.bitcast_convert_type(bc_ref[0], jnp.float32)
    gt = (bcf > hi).astype(jnp.float32)
    eq = (jnp.logical_and(bcf >= lo, bcf <= hi)).astype(jnp.float32)
    pref = jnp.dot(ltri_ref[...], eq, preferred_element_type=jnp.float32)
    cum = pref + pre
    keepcol = gt + eq * (cum <= m).astype(jnp.float32)
    o_ref[0] = x_ref[0] * keepcol


def kernel(x):
    B, L, C = x.shape
    K = max(1, int(L * (1.0 - _SPARSE_RATIO)))
    nc = L // _CL

    nb, nbcol = pl.pallas_call(
        functools.partial(_norm_body, L=L),
        grid=(B,),
        in_specs=[pl.BlockSpec((1, L, C), lambda b: (b, 0, 0))],
        out_specs=[
            pl.BlockSpec((1, L // 128, 128), lambda b: (b, 0, 0)),
            pl.BlockSpec((1, L, 1), lambda b: (b, 0, 0)),
        ],
        out_shape=[
            jax.ShapeDtypeStruct((B, L // 128, 128), jnp.int32),
            jax.ShapeDtypeStruct((B, L, 1), jnp.int32),
        ],
    )(x)

    info = plsc.get_sparse_core_info()
    NC = info.num_cores
    mesh = plsc.VectorSubcoreMesh(core_axis_name="c", subcore_axis_name="s")
    sc_fn = functools.partial(
        pl.kernel,
        out_type=jax.ShapeDtypeStruct((B, 32), jnp.float32),
        mesh=mesh,
        scratch_types=[
            pltpu.VMEM((L,), jnp.int32),
            pltpu.VMEM((L,), jnp.float32),
            pltpu.VMEM((32,), jnp.float32),
        ],
    )(functools.partial(_sc_body, K=K, L=L, B=B, NC=NC))
    aux = sc_fn(nb.reshape(B, L),
                jax.lax.bitcast_convert_type(nbcol.reshape(B, L),
                                             jnp.float32))

    ltri = jnp.tri(_CL, dtype=jnp.float32)
    return pl.pallas_call(
        _apply_body,
        grid=(B, nc),
        in_specs=[
            pl.BlockSpec((1, _CL, C), lambda b, c: (b, c, 0)),
            pl.BlockSpec((1, _CL, 1), lambda b, c: (b, c, 0)),
            pl.BlockSpec((1, 1, 32), lambda b, c: (b, 0, 0),
                         memory_space=pltpu.SMEM),
            pl.BlockSpec((_CL, _CL), lambda b, c: (0, 0)),
        ],
        out_specs=pl.BlockSpec((1, _CL, C), lambda b, c: (b, c, 0)),
        out_shape=jax.ShapeDtypeStruct((B, L, C), x.dtype),
    )(x, nbcol, aux.reshape(B, 1, 32), ltri)


# speedup vs baseline: 2.4359x; 2.4359x over previous
import functools

import jax
import jax.numpy as jnp
from jax.experimental import pallas as pl
from jax.experimental.pallas import tpu as pltpu

_SPARSE_RATIO = 0.5
_CL = 512


def _per_row_phase(buf, nb_ref, sq_ref, nbcol_ref, sc_ref, K, L):
    nc = L // _CL
    rows = _CL // 128
    for i in range(nc):
        part = buf[pl.ds(i * _CL, _CL), :]
        s = jnp.sum(part * part, axis=-1)
        nbcol_ref[pl.ds(i * _CL, _CL), :] = (
            jax.lax.bitcast_convert_type(s, jnp.int32)[:, None])
        s2 = s.reshape(rows, 128)
        nb_ref[pl.ds(i * rows, rows), :] = (
            jax.lax.bitcast_convert_type(jnp.sqrt(s2), jnp.int32))
        sq_ref[pl.ds(i * rows, rows), :] = (
            jax.lax.bitcast_convert_type(s2, jnp.int32))

    bits2 = nb_ref[...]

    def step(i, t):
        cand = jnp.bitwise_or(t, jnp.left_shift(jnp.int32(1), 30 - i))
        cnt = jnp.sum(jnp.where(bits2 >= cand, 1, 0))
        return jnp.where(cnt >= K, cand, t)

    T = jax.lax.fori_loop(0, 31, step, jnp.int32(0))
    eq2 = bits2 == T
    g = jnp.sum(jnp.where(bits2 > T, 1, 0))
    sq2 = sq_ref[...]
    sc_ref[0] = jnp.min(jnp.where(eq2, sq2, jnp.int32(2147483647)))
    sc_ref[1] = jnp.max(jnp.where(eq2, sq2, jnp.int32(-1)))
    sc_ref[2] = K - g
    riota = jax.lax.broadcasted_iota(jnp.int32, bits2.shape, 0)
    for cc in range(nc):
        sc_ref[3 + cc] = jnp.sum(
            jnp.where(jnp.logical_and(eq2, riota < cc * rows), 1, 0))


def _body(x_hbm, ltri_ref, o_ref, buf0, buf1, nb_ref, sq_ref, nbcol_ref,
          sc_ref, sem0, sem1, *, K: int, L: int, B: int):
    nc = L // _CL
    c = pl.program_id(1)
    b = pl.program_id(0)

    @pl.when(c == 0)
    def _row_setup():
        @pl.when(b == 0)
        def _():
            pltpu.make_async_copy(x_hbm.at[0], buf0, sem0).start()

        @pl.when(b % 2 == 0)
        def _():
            pltpu.make_async_copy(x_hbm.at[b], buf0, sem0).wait()

        @pl.when(b % 2 == 1)
        def _():
            pltpu.make_async_copy(x_hbm.at[b], buf1, sem1).wait()

        @pl.when(jnp.logical_and(b + 1 < B, (b + 1) % 2 == 0))
        def _():
            pltpu.make_async_copy(x_hbm.at[b + 1], buf0, sem0).start()

        @pl.when(jnp.logical_and(b + 1 < B, (b + 1) % 2 == 1))
        def _():
            pltpu.make_async_copy(x_hbm.at[b + 1], buf1, sem1).start()

        @pl.when(b % 2 == 0)
        def _():
            _per_row_phase(buf0, nb_ref, sq_ref, nbcol_ref, sc_ref, K, L)

        @pl.when(b % 2 == 1)
        def _():
            _per_row_phase(buf1, nb_ref, sq_ref, nbcol_ref, sc_ref, K, L)

    lo = sc_ref[0]
    hi = sc_ref[1]
    m = sc_ref[2]
    pre = sc_ref[3 + c]

    bc = nbcol_ref[pl.ds(c * _CL, _CL), :]
    gt = (bc > hi).astype(jnp.float32)
    eq = (jnp.logical_and(bc >= lo, bc <= hi)).astype(jnp.float32)
    pref = jnp.dot(ltri_ref[...], eq, preferred_element_type=jnp.float32)
    cum = pref + pre.astype(jnp.float32)
    keepcol = gt + eq * (cum <= m.astype(jnp.float32)).astype(jnp.float32)

    @pl.when(b % 2 == 0)
    def _():
        o_ref[0] = buf0[pl.ds(c * _CL, _CL), :] * keepcol

    @pl.when(b % 2 == 1)
    def _():
        o_ref[0] = buf1[pl.ds(c * _CL, _CL), :] * keepcol


def kernel(x):
    B, L, C = x.shape
    K = max(1, int(L * (1.0 - _SPARSE_RATIO)))
    nc = L // _CL

    ltri = jnp.tri(_CL, dtype=jnp.float32)

    return pl.pallas_call(
        functools.partial(_body, K=K, L=L, B=B),
        grid=(B, nc),
        in_specs=[
            pl.BlockSpec(memory_space=pl.ANY),
            pl.BlockSpec((_CL, _CL), lambda b, c: (0, 0)),
        ],
        out_specs=pl.BlockSpec((1, _CL, C), lambda b, c: (b, c, 0)),
        out_shape=jax.ShapeDtypeStruct((B, L, C), x.dtype),
        scratch_shapes=[
            pltpu.VMEM((L, C), jnp.float32),
            pltpu.VMEM((L, C), jnp.float32),
            pltpu.VMEM((L // 128, 128), jnp.int32),
            pltpu.VMEM((L // 128, 128), jnp.int32),
            pltpu.VMEM((L, 1), jnp.int32),
            pltpu.SMEM((3 + nc,), jnp.int32),
            pltpu.SemaphoreType.DMA,
            pltpu.SemaphoreType.DMA,
        ],
    )(x, ltri)
